# Initial kernel scaffold; baseline (speedup 1.0000x reference)
import functools

import jax
import jax.numpy as jnp
from jax import lax
from jax.experimental import pallas as pl
from jax.experimental.pallas import tpu as pltpu
from jax.experimental.pallas import tpu_sc as plsc

N = 10000
E = 320000
D = 128
NC = 2
NS = 16
NW = NC * NS
EPW = E // NW
CH = 80
NCHUNK = EPW // CH
RPT = N // NS


def _make_agg(with_deg: bool):
  mesh = plsc.VectorSubcoreMesh(
      core_axis_name="c", subcore_axis_name="s", num_cores=NC,
      num_subcores=NS)

  out_type = [jax.ShapeDtypeStruct((NC, N, D), jnp.float32)]
  if with_deg:
    out_type.append(jax.ShapeDtypeStruct((NC, N), jnp.float32))

  scratch = [
      pltpu.VMEM((CH,), jnp.int32),
      pltpu.VMEM((CH,), jnp.int32),
      pltpu.VMEM((CH, D), jnp.float32),
      pltpu.VMEM((RPT, D), jnp.float32),
      pltpu.VMEM_SHARED((N, D), jnp.float32),
      pltpu.SemaphoreType.DMA,
  ]
  if with_deg:
    scratch += [
        pltpu.VMEM((CH,), jnp.float32),
        pltpu.VMEM((2000,), jnp.float32),
        pltpu.VMEM_SHARED((N,), jnp.float32),
    ]

  def body(*refs):
    if with_deg:
      (x_hbm, src_hbm, dst_hbm, agg_out, deg_out,
       src_v, dst_v, rows_v, zrows_v, agg_sh, sem,
       ones_v, zvec_v, deg_sh) = refs
    else:
      (x_hbm, src_hbm, dst_hbm, agg_out,
       src_v, dst_v, rows_v, zrows_v, agg_sh, sem) = refs

    c = lax.axis_index("c")
    s = lax.axis_index("s")
    wid = s * NC + c

    def zrow_body(i, _):
      for j in range(D // 16):
        zrows_v[i, pl.ds(j * 16, 16)] = jnp.zeros((16,), jnp.float32)
      return 0
    lax.fori_loop(0, RPT, zrow_body, 0)
    row0 = s * RPT
    pltpu.sync_copy(zrows_v, agg_sh.at[pl.ds(row0, RPT)])
    if with_deg:
      def ones_body(i, _):
        ones_v[pl.ds(i * 16, 16)] = jnp.ones((16,), jnp.float32)
        return 0
      lax.fori_loop(0, CH // 16, ones_body, 0)
      def zvec_body(i, _):
        zvec_v[pl.ds(i * 16, 16)] = jnp.zeros((16,), jnp.float32)
        return 0
      lax.fori_loop(0, 2000 // 16, zvec_body, 0)
      @pl.when(s < 5)
      def _():
        pltpu.sync_copy(zvec_v, deg_sh.at[pl.ds(s * 2000, 2000)])
    plsc.subcore_barrier()

    ebase = wid * EPW

    def chunk_body(i, _):
      base = ebase + i * CH
      pltpu.sync_copy(src_hbm.at[pl.ds(base, CH)], src_v)
      pltpu.sync_copy(dst_hbm.at[pl.ds(base, CH)], dst_v)
      pltpu.async_copy(x_hbm.at[src_v], rows_v, sem).wait()
      pltpu.sync_copy(rows_v, agg_sh.at[dst_v], add=True)
      if with_deg:
        pltpu.sync_copy(ones_v, deg_sh.at[dst_v], add=True)
      return 0
    lax.fori_loop(0, NCHUNK, chunk_body, 0)
    plsc.subcore_barrier()

    for r in (0, 128, 256, 384):
      pltpu.sync_copy(agg_sh.at[pl.ds(row0 + r, 128)],
                      agg_out.at[c, pl.ds(row0 + r, 128), :])
    pltpu.sync_copy(agg_sh.at[pl.ds(row0 + 512, RPT - 512)],
                    agg_out.at[c, pl.ds(row0 + 512, RPT - 512), :])
    if with_deg:
      @pl.when(s < 5)
      def _():
        pltpu.sync_copy(deg_sh.at[pl.ds(s * 2000, 2000)],
                        deg_out.at[c, pl.ds(s * 2000, 2000)])

  return pl.kernel(body, out_type=out_type, mesh=mesh,
                   scratch_types=scratch)


_agg_with_deg = _make_agg(True)
_agg_no_deg = _make_agg(False)


def _dense(aggp, degp3, x, WlT, bl, WrT, relu):
  BN = 1000
  G = N // BN

  def body(aggp_ref, degp_ref, x_ref, wl_ref, bl_ref, wr_ref, o_ref):
    agg = aggp_ref[0] + aggp_ref[1]
    deg = degp_ref[0, 0, :] + degp_ref[0, 1, :]
    mean = agg / jnp.maximum(deg, 1.0)[:, None]
    acc = jnp.dot(mean, wl_ref[...], preferred_element_type=jnp.float32)
    acc = acc + bl_ref[...]
    acc = acc + jnp.dot(x_ref[...], wr_ref[...],
                        preferred_element_type=jnp.float32)
    if relu:
      acc = jnp.maximum(acc, 0.0)
    o_ref[...] = acc

  return pl.pallas_call(
      body,
      grid=(G,),
      in_specs=[
          pl.BlockSpec((NC, BN, D), lambda i: (0, i, 0)),
          pl.BlockSpec((1, NC, BN), lambda i: (i, 0, 0)),
          pl.BlockSpec((BN, D), lambda i: (i, 0)),
          pl.BlockSpec((D, D), lambda i: (0, 0)),
          pl.BlockSpec((1, D), lambda i: (0, 0)),
          pl.BlockSpec((D, D), lambda i: (0, 0)),
      ],
      out_specs=pl.BlockSpec((BN, D), lambda i: (i, 0)),
      out_shape=jax.ShapeDtypeStruct((N, D), jnp.float32),
  )(aggp, degp3, x, WlT, bl, WrT)


def kernel(x, edge_index, W1l, b1l, W1r, W2l, b2l, W2r):
  src = edge_index[0].astype(jnp.int32)
  dst = edge_index[1].astype(jnp.int32)

  aggp1, degp = _agg_with_deg(x, src, dst)
  degp3 = degp.reshape(NC, N // 1000, 1000).transpose(1, 0, 2)
  h = _dense(aggp1, degp3, x, W1l.T, b1l.reshape(1, D), W1r.T, relu=True)

  aggp2 = _agg_no_deg(h, src, dst)
  out = _dense(aggp2, degp3, h, W2l.T, b2l.reshape(1, D), W2r.T, relu=False)
  return out


# speedup vs baseline: 3.4789x; 3.4789x over previous
import jax
import jax.numpy as jnp
from jax import lax
from jax.experimental import pallas as pl
from jax.experimental.pallas import tpu as pltpu
from jax.experimental.pallas import tpu_sc as plsc

N = 10000
E = 320000
D = 128
DH = D // 2
NC = 2
NS = 16
EPT = E // NS
CH = 80
NCHUNK = EPT // CH
RPT = 632
RPT_LAST = N - (NS - 1) * RPT
NPAD = 10240
DPT = NPAD // NS


def _make_agg(with_deg: bool):
  mesh = plsc.VectorSubcoreMesh(
      core_axis_name="c", subcore_axis_name="s", num_cores=NC,
      num_subcores=NS)

  out_type = [jax.ShapeDtypeStruct((NC, N, DH), jnp.float32)]
  if with_deg:
    out_type.append(jax.ShapeDtypeStruct((NPAD,), jnp.float32))

  scratch = [
      pltpu.VMEM((CH,), jnp.int32),
      pltpu.VMEM((CH,), jnp.int32),
      pltpu.VMEM((CH, DH), jnp.float32),
      pltpu.VMEM((RPT, DH), jnp.float32),
      pltpu.VMEM_SHARED((N, DH), jnp.float32),
      pltpu.SemaphoreType.DMA,
  ]
  if with_deg:
    scratch += [
        pltpu.VMEM((CH,), jnp.float32),
        pltpu.VMEM((DPT,), jnp.float32),
        pltpu.VMEM_SHARED((NPAD,), jnp.float32),
    ]

  def body(*refs):
    if with_deg:
      (x2_hbm, src_hbm, dst_hbm, agg_out, deg_out,
       src_v, dst_v, rows_v, zrows_v, agg_sh, sem,
       ones_v, zvec_v, deg_sh) = refs
    else:
      (x2_hbm, src_hbm, dst_hbm, agg_out,
       src_v, dst_v, rows_v, zrows_v, agg_sh, sem) = refs

    c = lax.axis_index("c")
    s = lax.axis_index("s")

    def zrow_body(i, _):
      for j in range(DH // 16):
        zrows_v[i, pl.ds(j * 16, 16)] = jnp.zeros((16,), jnp.float32)
      return 0
    lax.fori_loop(0, RPT, zrow_body, 0)
    row0 = s * RPT
    @pl.when(s < NS - 1)
    def _():
      pltpu.sync_copy(zrows_v, agg_sh.at[pl.ds(row0, RPT)])
    @pl.when(s == NS - 1)
    def _():
      pltpu.sync_copy(zrows_v.at[pl.ds(0, RPT_LAST)],
                      agg_sh.at[pl.ds(row0, RPT_LAST)])
    if with_deg:
      def ones_body(i, _):
        ones_v[pl.ds(i * 16, 16)] = jnp.ones((16,), jnp.float32)
        return 0
      lax.fori_loop(0, CH // 16, ones_body, 0)
      def zvec_body(i, _):
        zvec_v[pl.ds(i * 16, 16)] = jnp.zeros((16,), jnp.float32)
        return 0
      lax.fori_loop(0, DPT // 16, zvec_body, 0)
      pltpu.sync_copy(zvec_v, deg_sh.at[pl.ds(s * DPT, DPT)])
    plsc.subcore_barrier()

    ebase = s * EPT

    def chunk_body(i, _):
      base = ebase + i * CH
      pltpu.sync_copy(src_hbm.at[pl.ds(base, CH)], src_v)
      pltpu.sync_copy(dst_hbm.at[pl.ds(base, CH)], dst_v)
      pltpu.async_copy(x2_hbm.at[c].at[src_v], rows_v, sem).wait()
      pltpu.sync_copy(rows_v, agg_sh.at[dst_v], add=True)
      if with_deg:
        @pl.when(c == 0)
        def _():
          pltpu.sync_copy(ones_v, deg_sh.at[dst_v], add=True)
      return 0
    lax.fori_loop(0, NCHUNK, chunk_body, 0)
    plsc.subcore_barrier()

    @pl.when(s < NS - 1)
    def _():
      pltpu.sync_copy(agg_sh.at[pl.ds(row0, RPT)],
                      agg_out.at[c, pl.ds(row0, RPT), :])
    @pl.when(s == NS - 1)
    def _():
      pltpu.sync_copy(agg_sh.at[pl.ds(row0, RPT_LAST)],
                      agg_out.at[c, pl.ds(row0, RPT_LAST), :])
    if with_deg:
      @pl.when(c == 0)
      def _():
        pltpu.sync_copy(deg_sh.at[pl.ds(s * DPT, DPT)],
                        deg_out.at[pl.ds(s * DPT, DPT)])

  return pl.kernel(body, out_type=out_type, mesh=mesh,
                   scratch_types=scratch,
                   compiler_params=pltpu.CompilerParams(
                       use_tc_tiling_on_sc=False))


_agg_with_deg = _make_agg(True)
_agg_no_deg = _make_agg(False)


def _dense(aggp, degp3, x, WlT, bl, WrT, relu):
  BN = 1000
  G = N // BN

  def body(aggp_ref, degp_ref, x_ref, wl_ref, bl_ref, wr_ref, o_ref):
    agg = jnp.concatenate([aggp_ref[0], aggp_ref[1]], axis=1)
    deg = degp_ref[0, 0, :]
    mean = agg / jnp.maximum(deg, 1.0)[:, None]
    acc = jnp.dot(mean, wl_ref[...], preferred_element_type=jnp.float32)
    acc = acc + bl_ref[...]
    acc = acc + jnp.dot(x_ref[...], wr_ref[...],
                        preferred_element_type=jnp.float32)
    if relu:
      acc = jnp.maximum(acc, 0.0)
    o_ref[...] = acc

  return pl.pallas_call(
      body,
      grid=(G,),
      in_specs=[
          pl.BlockSpec((NC, BN, DH), lambda i: (0, i, 0)),
          pl.BlockSpec((1, 1, BN), lambda i: (i, 0, 0)),
          pl.BlockSpec((BN, D), lambda i: (i, 0)),
          pl.BlockSpec((D, D), lambda i: (0, 0)),
          pl.BlockSpec((1, D), lambda i: (0, 0)),
          pl.BlockSpec((D, D), lambda i: (0, 0)),
      ],
      out_specs=pl.BlockSpec((BN, D), lambda i: (i, 0)),
      out_shape=jax.ShapeDtypeStruct((N, D), jnp.float32),
  )(aggp, degp3, x, WlT, bl, WrT)


def kernel(x, edge_index, W1l, b1l, W1r, W2l, b2l, W2r):
  src = edge_index[0].astype(jnp.int32)
  dst = edge_index[1].astype(jnp.int32)

  x2 = x.reshape(N, NC, DH).transpose(1, 0, 2)
  aggp1, deg = _agg_with_deg(x2, src, dst)
  degp3 = deg[:N].reshape(N // 1000, 1, 1000)
  h = _dense(aggp1, degp3, x, W1l.T, b1l.reshape(1, D), W1r.T, relu=True)

  h2 = h.reshape(N, NC, DH).transpose(1, 0, 2)
  aggp2, = _agg_no_deg(h2, src, dst)
  out = _dense(aggp2, degp3, h, W2l.T, b2l.reshape(1, D), W2r.T, relu=False)
  return out
